# Initial kernel scaffold; baseline (speedup 1.0000x reference)
import jax
import jax.numpy as jnp
from jax.experimental import pallas as pl


def kernel(buffer, buffer_targets, priorities, items, targets, new_priorities, store_idx, sample_idx):
    raise NotImplementedError("write your pallas kernel here")


# speedup vs baseline: 2.0844x; 2.0844x over previous
import functools

import jax
import jax.numpy as jnp
from jax import lax
from jax.experimental import pallas as pl
from jax.experimental.pallas import tpu as pltpu
from jax.experimental.pallas import tpu_sc as plsc

MAXN = 1_000_000
D = 64
B = 16384
NC = 2
NS = 16
L = 16
NW = NC * NS
RANGE = 31_264
MPAD = NW * RANGE
SUMCHUNK = 31_248
SUMTAIL = MAXN - NW * SUMCHUNK
Q = B // NW
NIDX = Q // 128

_mesh = plsc.VectorSubcoreMesh(core_axis_name="c", subcore_axis_name="s")


def _wid():
    return lax.axis_index("s") * NC + lax.axis_index("c")


@functools.partial(
    pl.kernel,
    out_type=(
        jax.ShapeDtypeStruct((MPAD,), jnp.int32),
        jax.ShapeDtypeStruct((NW, 128), jnp.float32),
    ),
    mesh=_mesh,
    compiler_params=pltpu.CompilerParams(needs_layout_passes=False, use_tc_tiling_on_sc=False),
    scratch_types=[
        pltpu.VMEM((B,), jnp.int32),
        pltpu.VMEM((RANGE,), jnp.int32),
        pltpu.VMEM((SUMCHUNK,), jnp.float32),
        pltpu.VMEM((SUMTAIL,), jnp.float32),
        pltpu.VMEM((128,), jnp.float32),
        pltpu.SemaphoreType.DMA,
        pltpu.SemaphoreType.DMA,
    ],
)
def _build_marker(sidx_hbm, pri_hbm, m_hbm, pp_hbm,
                  sidx_v, mark_v, pri_v, tail_v, acc_v, sem0, sem1):
    wid = _wid()
    base = wid * RANGE
    cp_idx = pltpu.async_copy(sidx_hbm, sidx_v, sem0)
    cp_pri = pltpu.async_copy(
        pri_hbm.at[pl.ds(wid * SUMCHUNK, SUMCHUNK)], pri_v, sem1)

    neg1 = jnp.full((L,), -1, jnp.int32)

    def init_body(i, c):
        mark_v[pl.ds(i * L, L)] = neg1
        return c
    lax.fori_loop(0, RANGE // L, init_body, 0)

    cp_idx.wait()
    iota = lax.iota(jnp.int32, L)

    def scatter_pass(check):
        def body(g, c):
            v = sidx_v[pl.ds(g * L, L)]
            rel = v - base
            msk = (rel >= 0) & (rel < RANGE)
            jv = iota + g * L
            if check:
                cur = plsc.load_gather(mark_v, [rel], mask=msk)
                wr = msk & (jv > cur)
            else:
                wr = msk
            plsc.store_scatter(mark_v, [rel], jv, mask=wr)
            return c
        lax.fori_loop(0, B // L, body, 0)

    scatter_pass(False)
    scatter_pass(True)
    scatter_pass(True)

    pltpu.sync_copy(mark_v, m_hbm.at[pl.ds(base, RANGE)])

    cp_pri.wait()
    zeros = jnp.zeros((L,), jnp.float32)
    for p in range(8):
        acc_v[pl.ds(p * L, L)] = zeros

    def sum_body(i, c):
        for p in range(4):
            sl = pl.ds(p * L, L)
            acc_v[sl] = acc_v[sl] + pri_v[pl.ds(i * 4 * L + p * L, L)]
        return c
    lax.fori_loop(0, SUMCHUNK // (4 * L), sum_body, 0)

    @pl.when(wid == NW - 1)
    def _():
        pltpu.sync_copy(pri_hbm.at[pl.ds(NW * SUMCHUNK, SUMTAIL)], tail_v)
        for t in range(SUMTAIL // L):
            acc_v[pl.ds(0, L)] = acc_v[pl.ds(0, L)] + tail_v[pl.ds(t * L, L)]

    pltpu.sync_copy(acc_v, pp_hbm.at[wid])


@functools.partial(
    pl.kernel,
    out_type=(
        jax.ShapeDtypeStruct((B, D), jnp.float32),
        jax.ShapeDtypeStruct((B,), jnp.float32),
        jax.ShapeDtypeStruct((B,), jnp.float32),
        jax.ShapeDtypeStruct((NW, 128), jnp.float32),
    ),
    mesh=_mesh,
    compiler_params=pltpu.CompilerParams(needs_layout_passes=False, use_tc_tiling_on_sc=False),
    scratch_types=[
        pltpu.VMEM((NIDX, 128), jnp.int32),
        pltpu.VMEM((NIDX, 128), jnp.int32),
        pltpu.VMEM((NIDX, 128), jnp.int32),
        pltpu.VMEM((Q,), jnp.int32),
        pltpu.VMEM((Q,), jnp.int32),
        pltpu.VMEM((Q,), jnp.float32),
        pltpu.VMEM((Q,), jnp.float32),
        pltpu.VMEM((Q,), jnp.float32),
        pltpu.VMEM((Q,), jnp.float32),
        pltpu.VMEM((Q,), jnp.float32),
        pltpu.VMEM((Q,), jnp.float32),
        pltpu.VMEM((Q,), jnp.float32),
        pltpu.VMEM((Q,), jnp.float32),
        pltpu.VMEM((Q, D), jnp.float32),
        pltpu.VMEM((Q, D), jnp.float32),
        pltpu.VMEM((128,), jnp.float32),
        pltpu.SemaphoreType.DMA,
        pltpu.SemaphoreType.DMA,
    ],
)
def _gather_select(m_hbm, buf_hbm, btgt_hbm, pri_hbm, items_hbm, tgt_hbm,
                   npri_hbm, sidx_hbm, qsel_hbm,
                   samples_hbm, stgt_hbm, spri_hbm, dp_hbm,
                   qidx, sidxc, wcbuf, wbuf, mw, pb, tb, tw, pw, pst, npc,
                   tsel, psel, brows, irows, dacc_v, semA, semB):
    wid = _wid()
    qbase = wid * Q

    w1 = []
    for c in range(NIDX):
        src = pl.ds(qbase + c * 128, 128)
        w1.append(pltpu.async_copy(qsel_hbm.at[src], qidx.at[c], semA))
        w1.append(pltpu.async_copy(sidx_hbm.at[src], sidxc.at[c], semA))
    w1.append(pltpu.async_copy(npri_hbm.at[pl.ds(qbase, Q)], npc, semA))
    for cp in w1:
        cp.wait()

    w2 = []
    for c in range(NIDX):
        sl = pl.ds(c * 128, 128)
        w2.append(pltpu.async_copy(m_hbm.at[qidx.at[c]], wbuf.at[sl], semB))
        w2.append(pltpu.async_copy(pri_hbm.at[qidx.at[c]], pb.at[sl], semB))
        w2.append(pltpu.async_copy(btgt_hbm.at[qidx.at[c]], tb.at[sl], semB))
        w2.append(pltpu.async_copy(buf_hbm.at[qidx.at[c]], brows.at[sl], semB))
        w2.append(pltpu.async_copy(m_hbm.at[sidxc.at[c]], mw.at[sl], semB))
        w2.append(pltpu.async_copy(pri_hbm.at[sidxc.at[c]], pst.at[sl], semB))
    for cp in w2:
        cp.wait()

    for c in range(NIDX):
        for k in range(128 // L):
            wv = wbuf[pl.ds(c * 128 + k * L, L)]
            wcbuf[c, pl.ds(k * L, L)] = jnp.maximum(wv, 0)

    w3 = []
    for c in range(NIDX):
        sl = pl.ds(c * 128, 128)
        w3.append(pltpu.async_copy(items_hbm.at[wcbuf.at[c]], irows.at[sl], semA))
        w3.append(pltpu.async_copy(tgt_hbm.at[wcbuf.at[c]], tw.at[sl], semA))
        w3.append(pltpu.async_copy(npri_hbm.at[wcbuf.at[c]], pw.at[sl], semA))
    for cp in w3:
        cp.wait()

    iota = lax.iota(jnp.int32, L)
    for p in range(8):
        dacc_v[pl.ds(p * L, L)] = jnp.zeros((L,), jnp.float32)
    for t in range(Q // L):
        sl = pl.ds(t * L, L)
        wv = wbuf[sl]
        hit = wv >= 0
        tsel[sl] = jnp.where(hit, tw[sl], tb[sl])
        psel[sl] = jnp.where(hit, pw[sl], pb[sl])
        jvq = iota + (qbase + t * L)
        win = mw[sl] == jvq
        dv = pl.ds(0, L)
        dacc_v[dv] = dacc_v[dv] + jnp.where(win, npc[sl] - pst[sl], 0.0)

    def merge_body(t, c):
        wv = wbuf[pl.ds(t * L, L)]
        for lane in range(L):
            @pl.when(wv[lane] >= 0)
            def _(lane=lane):
                q = t * L + lane
                for d in range(D // L):
                    brows[q, pl.ds(d * L, L)] = irows[q, pl.ds(d * L, L)]
        return c
    lax.fori_loop(0, Q // L, merge_body, 0)

    pltpu.sync_copy(brows, samples_hbm.at[pl.ds(qbase, Q)])
    pltpu.sync_copy(tsel, stgt_hbm.at[pl.ds(qbase, Q)])
    pltpu.sync_copy(psel, spri_hbm.at[pl.ds(qbase, Q)])
    pltpu.sync_copy(dacc_v, dp_hbm.at[wid])


def _norm_body(spri_ref, pp_ref, dp_ref, out_ref):
    tot = jnp.sum(pp_ref[...]) + jnp.sum(dp_ref[...])
    out_ref[...] = spri_ref[...] / tot


_normalize = pl.pallas_call(
    _norm_body,
    out_shape=jax.ShapeDtypeStruct((128, 128), jnp.float32),
)


def kernel(buffer, buffer_targets, priorities, items, targets,
           new_priorities, store_idx, sample_idx):
    m, pp = _build_marker(store_idx, priorities)
    samples, stgt, spri, dp = _gather_select(
        m, buffer, buffer_targets, priorities, items, targets,
        new_priorities, store_idx, sample_idx)
    probs = _normalize(spri.reshape(128, 128), pp, dp)
    return samples, stgt, probs.reshape(B), sample_idx
